# Initial kernel scaffold; baseline (speedup 1.0000x reference)
import jax
import jax.numpy as jnp
from jax.experimental import pallas as pl


def kernel(cam_feat, ego2cam, img_shape, bev_planes, W, b):
    raise NotImplementedError("write your pallas kernel here")


# speedup vs baseline: 128.3728x; 128.3728x over previous
import functools

import jax
import jax.numpy as jnp
from jax import lax
from jax.experimental import pallas as pl
from jax.experimental.pallas import tpu as pltpu
from jax.experimental.pallas import tpu_sc as plsc

F32 = jnp.float32
I32 = jnp.int32

LVL = 3
BH = 200
BW = 200
NS = BH * BW
SPAD = 40960
IH, IW = 32, 88
PH, PW = IH + 2, IW + 2
TROWS = PH * PW
TPAD = 3072
C = 128

NWORK = 32
KPTS = 32
NCHUNK = LVL * SPAD // KPTS
CHUNK_PER_W = NCHUNK // NWORK

CT = 4


def _proj_kernel(p9_ref, m_ref, sh_ref, base_ref, w_ref):
    def rb(v):
        return v.astype(jnp.bfloat16).astype(F32)

    x = rb(p9_ref[0:3, :])
    y = rb(p9_ref[3:6, :])
    z = rb(p9_ref[6:9, :])

    def cam_row(i):
        m0 = rb(m_ref[i, 0])
        m1 = rb(m_ref[i, 1])
        m2 = rb(m_ref[i, 2])
        m3 = rb(m_ref[i, 3])
        return ((m0 * x + m1 * y) + m2 * z) + m3

    cx = cam_row(0)
    cy = cam_row(1)
    cz = cam_row(2)
    eps = jnp.float32(1e-9)
    wf = sh_ref[1].astype(F32)
    hf = sh_ref[0].astype(F32)
    def div(a, bb):
        r = 1.0 / bb
        r = r * (2.0 - bb * r)
        q = a * r
        return q + (a - q * bb) * r

    xn = (div(div(cx, cz) + eps, wf) - 0.5) * 2.0
    yn = (div(div(cy, cz) + eps, hf) - 0.5) * 2.0
    mask = (cz > eps) & (xn > -1.0) & (xn < 1.0) & (yn > -1.0) & (yn < 1.0)
    xp = ((xn + 1.0) * IW - 1.0) * 0.5
    yp = ((yn + 1.0) * IH - 1.0) * 0.5
    x0f = jnp.floor(xp)
    y0f = jnp.floor(yp)
    fx = xp - x0f
    fy = yp - y0f
    zero = jnp.zeros_like(fx)
    wa = jnp.where(mask, (1.0 - fx) * (1.0 - fy), zero)
    wc = jnp.where(mask, fx * (1.0 - fy), zero)
    wb = jnp.where(mask, (1.0 - fx) * fy, zero)
    wd = jnp.where(mask, fx * fy, zero)
    x0i = jnp.clip(x0f + 1.0, 0.0, float(IW)).astype(I32)
    y0i = jnp.clip(y0f + 1.0, 0.0, float(IH)).astype(I32)
    base = jnp.clip(y0i * PW + x0i, 0, (PH - 2) * PW + (PW - 2))
    base_ref[...] = base
    w_ref[:, 0, :] = wa
    w_ref[:, 1, :] = wc
    w_ref[:, 2, :] = wb
    w_ref[:, 3, :] = wd


def _proj_stage(p9, m, sh):
    nsteps = 20
    blk = SPAD // nsteps
    return pl.pallas_call(
        _proj_kernel,
        grid=(nsteps,),
        in_specs=[
            pl.BlockSpec((9, blk), lambda i: (0, i)),
            pl.BlockSpec((4, 4), lambda i: (0, 0), memory_space=pltpu.SMEM),
            pl.BlockSpec((2,), lambda i: (0,), memory_space=pltpu.SMEM),
        ],
        out_specs=[
            pl.BlockSpec((3, blk), lambda i: (0, i)),
            pl.BlockSpec((3, 4, blk), lambda i: (0, 0, i)),
        ],
        out_shape=[
            jax.ShapeDtypeStruct((LVL, SPAD), I32),
            jax.ShapeDtypeStruct((LVL, 4, SPAD), F32),
        ],
    )(p9, m, sh)


def _sc_body(table, base3, w4, out, idxv, rowsv, bsev, wv, outv, sem):
    ci = lax.axis_index("c")
    si = lax.axis_index("s")
    wid = si * 2 + ci
    dnums = lax.GatherDimensionNumbers(
        offset_dims=(), collapsed_slice_dims=(0,), start_index_map=(0,))

    def bcast_lane(vec, lane):
        idx = jnp.full((16, 1), lane, I32)
        return lax.gather(vec, idx, dnums, (1,),
                          mode=lax.GatherScatterMode.PROMISE_IN_BOUNDS)

    def body(i, carry):
        p0 = (wid * CHUNK_PER_W + i) * KPTS
        lvl = p0 // SPAD
        s0 = p0 - lvl * SPAD
        pltpu.sync_copy(base3.at[lvl, pl.ds(s0, KPTS)], bsev)
        for k in range(4):
            pltpu.sync_copy(w4.at[lvl, k, pl.ds(s0, KPTS)], wv.at[k])
        for j in range(2):
            bv = bsev[pl.ds(j * 16, 16)]
            for k, off in enumerate((0, 1, PW, PW + 1)):
                idxv[pl.ds(k * KPTS + j * 16, 16)] = bv + off
        pltpu.async_copy(table.at[idxv], rowsv, sem).wait()
        wvecs = [[wv[k, pl.ds(0, 16)], wv[k, pl.ds(16, 16)]] for k in range(4)]
        for p in range(KPTS):
            g, l = divmod(p, 16)
            wks = [bcast_lane(wvecs[k][g], l) for k in range(4)]
            for c8 in range(8):
                acc = None
                for k in range(4):
                    term = wks[k] * rowsv[k * KPTS + p, pl.ds(c8 * 16, 16)]
                    acc = term if acc is None else acc + term
                outv[p, pl.ds(c8 * 16, 16)] = acc
        pltpu.sync_copy(outv, out.at[lvl, pl.ds(s0, KPTS), :])
        return carry

    lax.fori_loop(0, CHUNK_PER_W, body, 0)


def _sc_gather(table, base3, w4):
    mesh = plsc.VectorSubcoreMesh(core_axis_name="c", subcore_axis_name="s")
    fn = pl.kernel(
        _sc_body,
        out_type=jax.ShapeDtypeStruct((LVL, SPAD, C), F32),
        mesh=mesh,
        scratch_types=[
            pltpu.VMEM((4 * KPTS,), I32),
            pltpu.VMEM((4 * KPTS, C), F32),
            pltpu.VMEM((KPTS,), I32),
            pltpu.VMEM((4, KPTS), F32),
            pltpu.VMEM((KPTS, C), F32),
            pltpu.SemaphoreType.DMA,
        ],
    )
    return fn(table, base3, w4)


def _conv_kernel(fa_ref, fb_ref, ga_ref, gb_ref, wp_ref, wg_ref, b_ref, out_ref):
    win = jnp.concatenate([fa_ref[...], fb_ref[0:2]], axis=0)
    gwin = jnp.concatenate([ga_ref[...], gb_ref[0:2]], axis=0)
    acc = None
    for dy in range(3):
        for dx in range(3):
            sub = win[dy:dy + CT, dx:dx + BW, :].reshape(CT * BW, LVL * C)
            t = lax.dot_general(sub, wp_ref[dy * 3 + dx],
                                (((1,), (0,)), ((), ())),
                                preferred_element_type=F32)
            subg = gwin[dy:dy + CT, dx:dx + BW, :].reshape(CT * BW, 16)
            t = t + lax.dot_general(subg, wg_ref[dy * 3 + dx],
                                    (((1,), (0,)), ((), ())),
                                    preferred_element_type=F32)
            acc = t if acc is None else acc + t
    out_ref[...] = (acc + b_ref[...]).reshape(CT, BW, C)


def _conv_stage(fp, gp, wp9, wg9, bias):
    nsteps = BH // CT
    return pl.pallas_call(
        _conv_kernel,
        grid=(nsteps,),
        in_specs=[
            pl.BlockSpec((CT, BW + 2, LVL * C), lambda t: (t, 0, 0)),
            pl.BlockSpec((CT, BW + 2, LVL * C), lambda t: (t + 1, 0, 0)),
            pl.BlockSpec((CT, BW + 2, 16), lambda t: (t, 0, 0)),
            pl.BlockSpec((CT, BW + 2, 16), lambda t: (t + 1, 0, 0)),
            pl.BlockSpec((9, LVL * C, C), lambda t: (0, 0, 0)),
            pl.BlockSpec((9, 16, C), lambda t: (0, 0, 0)),
            pl.BlockSpec((1, C), lambda t: (0, 0)),
        ],
        out_specs=pl.BlockSpec((CT, BW, C), lambda t: (t, 0, 0)),
        out_shape=jax.ShapeDtypeStruct((BH, BW, C), F32),
    )(fp, fp, gp, gp, wp9, wg9, bias)


def kernel(cam_feat, ego2cam, img_shape, bev_planes, W, b):
    m = ego2cam[0, 0].astype(F32)
    p9 = bev_planes.transpose(3, 0, 1, 2).reshape(9, NS)
    p9 = jnp.pad(p9, ((0, 0), (0, SPAD - NS)))
    im_p = jnp.pad(cam_feat[0], ((0, 0), (1, 1), (1, 1)))
    table = im_p.transpose(1, 2, 0).reshape(TROWS, C)
    table = jnp.pad(table, ((0, TPAD - TROWS), (0, 0)))

    base3, w4 = _proj_stage(p9, m, img_shape)

    proj = _sc_gather(table, base3, w4)

    feat = proj[:, :NS, :].reshape(LVL, BH, BW, C)
    feat = feat.transpose(1, 2, 0, 3).reshape(BH, BW, LVL * C)
    fp = jnp.pad(feat, ((1, CT - 1), (1, 1), (0, 0)))
    g3 = bev_planes.transpose(1, 2, 0, 3).reshape(BH, BW, 9)
    gp = jnp.pad(g3, ((1, CT - 1), (1, 1), (0, 7)))

    wp9 = W[:, :LVL * C].reshape(C, C, LVL, 3, 3)
    wp9 = wp9.transpose(3, 4, 2, 1, 0).reshape(9, LVL * C, C)
    wg9 = W[:, LVL * C:].reshape(C, 3, LVL, 3, 3)
    wg9 = wg9.transpose(3, 4, 2, 1, 0).reshape(3, 3, 9, C)
    wg9 = jnp.pad(wg9, ((0, 0), (0, 0), (0, 7), (0, 0))).reshape(9, 16, C)
    bias = b.reshape(1, C)

    out = _conv_stage(fp, gp, wp9, wg9, bias)
    return out.transpose(2, 0, 1)[None]


# speedup vs baseline: 128.5567x; 1.0014x over previous
import functools

import jax
import jax.numpy as jnp
from jax import lax
from jax.experimental import pallas as pl
from jax.experimental.pallas import tpu as pltpu
from jax.experimental.pallas import tpu_sc as plsc

F32 = jnp.float32
I32 = jnp.int32

LVL = 3
BH = 200
BW = 200
NS = BH * BW
SPAD = 40960
IH, IW = 32, 88
PH, PW = IH + 2, IW + 2
TROWS = PH * PW
TPAD = 3072
C = 128

NWORK = 32
KPTS = 32
NCHUNK = LVL * SPAD // KPTS
CHUNK_PER_W = NCHUNK // NWORK

CT = 4


def _proj_kernel(p9_ref, m_ref, sh_ref, base_ref, w_ref):
    def rb(v):
        return v.astype(jnp.bfloat16).astype(F32)

    x = rb(p9_ref[0:3, :])
    y = rb(p9_ref[3:6, :])
    z = rb(p9_ref[6:9, :])

    def cam_row(i):
        m0 = rb(m_ref[i, 0])
        m1 = rb(m_ref[i, 1])
        m2 = rb(m_ref[i, 2])
        m3 = rb(m_ref[i, 3])
        return ((m0 * x + m1 * y) + m2 * z) + m3

    cx = cam_row(0)
    cy = cam_row(1)
    cz = cam_row(2)
    eps = jnp.float32(1e-9)
    wf = sh_ref[1].astype(F32)
    hf = sh_ref[0].astype(F32)
    def div(a, bb):
        r = 1.0 / bb
        r = r * (2.0 - bb * r)
        q = a * r
        return q + (a - q * bb) * r

    xn = (div(div(cx, cz) + eps, wf) - 0.5) * 2.0
    yn = (div(div(cy, cz) + eps, hf) - 0.5) * 2.0
    mask = (cz > eps) & (xn > -1.0) & (xn < 1.0) & (yn > -1.0) & (yn < 1.0)
    xp = ((xn + 1.0) * IW - 1.0) * 0.5
    yp = ((yn + 1.0) * IH - 1.0) * 0.5
    x0f = jnp.floor(xp)
    y0f = jnp.floor(yp)
    fx = xp - x0f
    fy = yp - y0f
    zero = jnp.zeros_like(fx)
    wa = jnp.where(mask, (1.0 - fx) * (1.0 - fy), zero)
    wc = jnp.where(mask, fx * (1.0 - fy), zero)
    wb = jnp.where(mask, (1.0 - fx) * fy, zero)
    wd = jnp.where(mask, fx * fy, zero)
    x0i = jnp.clip(x0f + 1.0, 0.0, float(IW)).astype(I32)
    y0i = jnp.clip(y0f + 1.0, 0.0, float(IH)).astype(I32)
    base = jnp.clip(y0i * PW + x0i, 0, (PH - 2) * PW + (PW - 2))
    base_ref[...] = base
    w_ref[0] = wa
    w_ref[1] = wc
    w_ref[2] = wb
    w_ref[3] = wd


def _proj_stage(p9, m, sh):
    nsteps = 20
    blk = SPAD // nsteps
    return pl.pallas_call(
        _proj_kernel,
        grid=(nsteps,),
        in_specs=[
            pl.BlockSpec((9, blk), lambda i: (0, i)),
            pl.BlockSpec((4, 4), lambda i: (0, 0), memory_space=pltpu.SMEM),
            pl.BlockSpec((2,), lambda i: (0,), memory_space=pltpu.SMEM),
        ],
        out_specs=[
            pl.BlockSpec((3, blk), lambda i: (0, i)),
            pl.BlockSpec((4, 3, blk), lambda i: (0, 0, i)),
        ],
        out_shape=[
            jax.ShapeDtypeStruct((LVL, SPAD), I32),
            jax.ShapeDtypeStruct((4, LVL, SPAD), F32),
        ],
    )(p9, m, sh)


PPW = LVL * SPAD // NWORK


def _sc_body(table, baseq, wq, out, idxv, rowsv, ball, wall, outv, sem):
    ci = lax.axis_index("c")
    si = lax.axis_index("s")
    wid = si * 2 + ci
    dnums = lax.GatherDimensionNumbers(
        offset_dims=(), collapsed_slice_dims=(0,), start_index_map=(0,))

    def bcast_lane(vec, lane):
        idx = jnp.full((16, 1), lane, I32)
        return lax.gather(vec, idx, dnums, (1,),
                          mode=lax.GatherScatterMode.PROMISE_IN_BOUNDS)

    pltpu.sync_copy(baseq.at[pl.ds(wid * PPW, PPW)], ball)
    for k in range(4):
        pltpu.sync_copy(wq.at[k, pl.ds(wid * PPW, PPW)], wall.at[k])

    def body(i, carry):
        p0 = wid * PPW + i * KPTS
        lvl = p0 // SPAD
        s0 = p0 - lvl * SPAD
        for j in range(2):
            bv = ball[pl.ds(i * KPTS + j * 16, 16)]
            for k, off in enumerate((0, 1, PW, PW + 1)):
                idxv[pl.ds(k * KPTS + j * 16, 16)] = bv + off
        pltpu.async_copy(table.at[idxv], rowsv, sem).wait()
        for p in range(KPTS):
            g, l = divmod(p, 16)
            wks = [bcast_lane(wall[k, pl.ds(i * KPTS + g * 16, 16)], l)
                   for k in range(4)]
            for c8 in range(8):
                acc = None
                for k in range(4):
                    term = wks[k] * rowsv[k * KPTS + p, pl.ds(c8 * 16, 16)]
                    acc = term if acc is None else acc + term
                outv[p, pl.ds(c8 * 16, 16)] = acc
        pltpu.sync_copy(outv, out.at[lvl, pl.ds(s0, KPTS), :])
        return carry

    lax.fori_loop(0, CHUNK_PER_W, body, 0)


def _sc_gather(table, base3, w4):
    mesh = plsc.VectorSubcoreMesh(core_axis_name="c", subcore_axis_name="s")
    fn = pl.kernel(
        _sc_body,
        out_type=jax.ShapeDtypeStruct((LVL, SPAD, C), F32),
        mesh=mesh,
        scratch_types=[
            pltpu.VMEM((4 * KPTS,), I32),
            pltpu.VMEM((4 * KPTS, C), F32),
            pltpu.VMEM((PPW,), I32),
            pltpu.VMEM((4, PPW), F32),
            pltpu.VMEM((KPTS, C), F32),
            pltpu.SemaphoreType.DMA,
        ],
    )
    return fn(table, base3, w4)


def _conv_kernel(fa_ref, fb_ref, ga_ref, gb_ref, wp_ref, wg_ref, b_ref, out_ref):
    win = jnp.concatenate([fa_ref[...], fb_ref[0:2]], axis=0)
    gwin = jnp.concatenate([ga_ref[...], gb_ref[0:2]], axis=0)
    acc = None
    for dy in range(3):
        for dx in range(3):
            sub = win[dy:dy + CT, dx:dx + BW, :].reshape(CT * BW, LVL * C)
            t = lax.dot_general(sub, wp_ref[dy * 3 + dx],
                                (((1,), (0,)), ((), ())),
                                preferred_element_type=F32)
            subg = gwin[dy:dy + CT, dx:dx + BW, :].reshape(CT * BW, 16)
            t = t + lax.dot_general(subg, wg_ref[dy * 3 + dx],
                                    (((1,), (0,)), ((), ())),
                                    preferred_element_type=F32)
            acc = t if acc is None else acc + t
    out_ref[...] = (acc + b_ref[...]).reshape(CT, BW, C)


def _conv_stage(fp, gp, wp9, wg9, bias):
    nsteps = BH // CT
    return pl.pallas_call(
        _conv_kernel,
        grid=(nsteps,),
        in_specs=[
            pl.BlockSpec((CT, BW + 2, LVL * C), lambda t: (t, 0, 0)),
            pl.BlockSpec((CT, BW + 2, LVL * C), lambda t: (t + 1, 0, 0)),
            pl.BlockSpec((CT, BW + 2, 16), lambda t: (t, 0, 0)),
            pl.BlockSpec((CT, BW + 2, 16), lambda t: (t + 1, 0, 0)),
            pl.BlockSpec((9, LVL * C, C), lambda t: (0, 0, 0)),
            pl.BlockSpec((9, 16, C), lambda t: (0, 0, 0)),
            pl.BlockSpec((1, C), lambda t: (0, 0)),
        ],
        out_specs=pl.BlockSpec((CT, BW, C), lambda t: (t, 0, 0)),
        out_shape=jax.ShapeDtypeStruct((BH, BW, C), F32),
    )(fp, fp, gp, gp, wp9, wg9, bias)


def kernel(cam_feat, ego2cam, img_shape, bev_planes, W, b):
    m = ego2cam[0, 0].astype(F32)
    p9 = bev_planes.transpose(3, 0, 1, 2).reshape(9, NS)
    p9 = jnp.pad(p9, ((0, 0), (0, SPAD - NS)))
    im_p = jnp.pad(cam_feat[0], ((0, 0), (1, 1), (1, 1)))
    table = im_p.transpose(1, 2, 0).reshape(TROWS, C)
    table = jnp.pad(table, ((0, TPAD - TROWS), (0, 0)))

    base3, w4 = _proj_stage(p9, m, img_shape)

    proj = _sc_gather(table, base3.reshape(-1), w4.reshape(4, -1))

    feat = proj[:, :NS, :].reshape(LVL, BH, BW, C)
    feat = feat.transpose(1, 2, 0, 3).reshape(BH, BW, LVL * C)
    fp = jnp.pad(feat, ((1, CT - 1), (1, 1), (0, 0)))
    g3 = bev_planes.transpose(1, 2, 0, 3).reshape(BH, BW, 9)
    gp = jnp.pad(g3, ((1, CT - 1), (1, 1), (0, 7)))

    wp9 = W[:, :LVL * C].reshape(C, C, LVL, 3, 3)
    wp9 = wp9.transpose(3, 4, 2, 1, 0).reshape(9, LVL * C, C)
    wg9 = W[:, LVL * C:].reshape(C, 3, LVL, 3, 3)
    wg9 = wg9.transpose(3, 4, 2, 1, 0).reshape(3, 3, 9, C)
    wg9 = jnp.pad(wg9, ((0, 0), (0, 0), (0, 7), (0, 0))).reshape(9, 16, C)
    bias = b.reshape(1, C)

    out = _conv_stage(fp, gp, wp9, wg9, bias)
    return out.transpose(2, 0, 1)[None]
